# baseline (speedup 1.0000x reference)
import math

import jax
import jax.numpy as jnp
from jax import lax
from jax.experimental import pallas as pl
from jax.experimental.pallas import tpu as pltpu
from jax.experimental.pallas import tpu_sc as plsc

_NUM_LEVELS = 16
_FEATS = 2
_TABLE = 2 ** 19
_MASK = _TABLE - 1
_MIN_RES = 16
_MAX_RES = 512
_P1 = 1540863
_P2 = 1256879
_P3 = 1957123

_GROWTH = math.exp(math.log(_MAX_RES / _MIN_RES) / (_NUM_LEVELS - 1))
_RES = [int(math.floor(_MIN_RES * _GROWTH ** l + 1e-06)) for l in range(_NUM_LEVELS)]

_CORNERS = [(ox, oy, oz) for ox in (0, 1) for oy in (0, 1) for oz in (0, 1)]

_NC = 2
_NS = 16
_NW = _NC * _NS

_C = 128
_G = _C // 16
_R = _C * 8 * _NUM_LEVELS
_OUTW = _NUM_LEVELS * _FEATS


def _vdup(v, idx_const):
    dn = lax.GatherDimensionNumbers(
        offset_dims=(), collapsed_slice_dims=(0,), start_index_map=(0,))
    return lax.gather(v, idx_const[:, None], dn, (1,),
                      mode=lax.GatherScatterMode.PROMISE_IN_BOUNDS)


def _hash_grid_sc(x2d, tab2d, n_points):
    per_w = n_points // _NW
    n_chunks = per_w // _C

    mesh = plsc.VectorSubcoreMesh(core_axis_name="c", subcore_axis_name="s")

    def body(x_hbm, tab_hbm, res_hbm, out_hbm, x_v, res_v, idx_v, w_v, rows_v,
             outb_v, sem):
        wid = lax.axis_index("s") * _NC + lax.axis_index("c")
        wbase = wid * per_w

        pltpu.sync_copy(res_hbm, res_v)
        resvec = res_v[...]
        iota = lax.iota(jnp.int32, 16)
        lvec = iota * _TABLE
        dup0 = iota >> 1
        dup1 = (iota >> 1) + 8
        fbit = iota & 1

        def chunk_body(c, carry):
            pbase = wbase + c * _C

            pltpu.sync_copy(x_hbm.at[:, pl.ds(pbase, _C)], x_v)

            def index_group(g, _):
                xv = jnp.clip(x_v[0, pl.ds(g * 16, 16)], 0.0, 1.0)
                yv = jnp.clip(x_v[1, pl.ds(g * 16, 16)], 0.0, 1.0)
                zv = jnp.clip(x_v[2, pl.ds(g * 16, 16)], 0.0, 1.0)

                def index_point(k, _):
                    p = g * 16 + k
                    lane = jnp.full((16,), k, dtype=jnp.int32)
                    xb = _vdup(xv, lane)
                    yb = _vdup(yv, lane)
                    zb = _vdup(zv, lane)
                    px = xb * resvec
                    py = yb * resvec
                    pz = zb * resvec
                    ix0 = px.astype(jnp.int32)
                    iy0 = py.astype(jnp.int32)
                    iz0 = pz.astype(jnp.int32)
                    fx = px - ix0.astype(jnp.float32)
                    fy = py - iy0.astype(jnp.float32)
                    fz = pz - iz0.astype(jnp.float32)
                    hx = (ix0 * _P1, ix0 * _P1 + _P1)
                    hy = (iy0 * _P2, iy0 * _P2 + _P2)
                    hz = (iz0 * _P3, iz0 * _P3 + _P3)
                    wx = (1.0 - fx, fx)
                    wy = (1.0 - fy, fy)
                    wz = (1.0 - fz, fz)
                    for j, (ox, oy, oz) in enumerate(_CORNERS):
                        h = (hx[ox] ^ hy[oy]) ^ hz[oz]
                        elem = (((h & _MASK) + lvec) << 1)
                        wj = (wx[ox] * wy[oy]) * wz[oz]
                        off = (p * 8 + j) * 16
                        idx_v[pl.ds(2 * off, 16)] = _vdup(elem, dup0) | fbit
                        idx_v[pl.ds(2 * off + 16, 16)] = _vdup(elem, dup1) | fbit
                        w_v[pl.ds(off, 16)] = wj
                    return 0

                lax.fori_loop(0, 16, index_point, 0)
                return 0

            lax.fori_loop(0, _G, index_group, 0)

            pltpu.async_copy(tab_hbm.at[idx_v], rows_v, sem).wait()

            def acc_point(p, _):
                acc0 = None
                acc1 = None
                for j in range(8):
                    off = (p * 8 + j) * 16
                    w = w_v[pl.ds(off, 16)]
                    w0 = _vdup(w, dup0)
                    w1 = _vdup(w, dup1)
                    r0 = rows_v[pl.ds(2 * off, 16)]
                    r1 = rows_v[pl.ds(2 * off + 16, 16)]
                    if acc0 is None:
                        acc0 = w0 * r0
                        acc1 = w1 * r1
                    else:
                        acc0 = acc0 + w0 * r0
                        acc1 = acc1 + w1 * r1
                outb_v[pl.ds(p * _OUTW, 16)] = acc0
                outb_v[pl.ds(p * _OUTW + 16, 16)] = acc1
                return 0

            lax.fori_loop(0, _C, acc_point, 0)

            pltpu.sync_copy(outb_v, out_hbm.at[pl.ds(pbase * _OUTW, _C * _OUTW)])
            return carry

        lax.fori_loop(0, n_chunks, chunk_body, 0)

    kern = pl.kernel(
        body,
        out_type=jax.ShapeDtypeStruct((n_points * _OUTW,), jnp.float32),
        mesh=mesh,
        scratch_types=[
            pltpu.VMEM((3, _C), jnp.float32),
            pltpu.VMEM((16,), jnp.float32),
            pltpu.VMEM((2 * _R,), jnp.int32),
            pltpu.VMEM((_R,), jnp.float32),
            pltpu.VMEM((2 * _R,), jnp.float32),
            pltpu.VMEM((_C * _OUTW,), jnp.float32),
            pltpu.SemaphoreType.DMA,
        ],
        compiler_params=pltpu.CompilerParams(needs_layout_passes=False),
    )
    res_arr = jnp.asarray([float(r) for r in _RES], dtype=jnp.float32)
    return kern(x2d, tab2d, res_arr)


def kernel(x01, tables):
    n = x01.shape[0]
    x2d = x01.T
    tab_flat = tables.reshape(-1)
    out = _hash_grid_sc(x2d, tab_flat, n)
    return out.reshape(n, _OUTW)
